# baseline (speedup 1.0000x reference)
import functools

import numpy as np
import jax
import jax.numpy as jnp
from jax import lax
from jax.experimental import pallas as pl
from jax.experimental.pallas import tpu as pltpu
from jax.experimental.pallas import tpu_sc as plsc

_N_ROWS = 16384
_SEQ_LEN = 200
_NA = 25
_W = _SEQ_LEN * _NA
_N_POS = _N_ROWS * _SEQ_LEN


_TC_BLOCK_ROWS = 256

_JDX = np.arange(_W) // _NA
_REP_NP = (_JDX[None, :] == np.arange(_SEQ_LEN)[:, None]).astype(np.float32)
_REP_BF16 = _REP_NP.astype(jnp.bfloat16)


def _tc_onehot_kernel(seq_ref, rep_ref, out_ref):
  qi = lax.broadcasted_iota(jnp.int32, (_TC_BLOCK_ROWS, _W), 1)
  seq_bf = seq_ref[...].astype(jnp.float32).astype(jnp.bfloat16)
  sval = jnp.dot(seq_bf, rep_ref[...], preferred_element_type=jnp.float32)
  jdx = (qi * 10486) >> 18
  kdx = (qi - _NA * jdx).astype(jnp.float32)
  out_ref[...] = jnp.where(kdx == sval, 1.0, 0.0)


_NC = 2
_NS = 16
_NW = _NC * _NS

_ROWS_PER_W = _N_ROWS // _NW
_CHUNK_ROWS = 16
_N_CHUNKS = _ROWS_PER_W // _CHUNK_ROWS
_CHUNK_POS = _CHUNK_ROWS * _SEQ_LEN
_N_GRP = _CHUNK_POS // 16
_NSLOT = 4

_KS0 = np.uint32(0)
_KS1 = np.uint32(42)
_KS2 = np.uint32(0x1BD11BDA ^ 42)
_ROTS = ((13, 15, 26, 6), (17, 29, 16, 24))
_INJECT = (
    (_KS1, np.uint32(_KS2 + np.uint32(1))),
    (_KS2, np.uint32(_KS0 + np.uint32(2))),
    (_KS0, np.uint32(_KS1 + np.uint32(3))),
    (_KS1, np.uint32(_KS2 + np.uint32(4))),
    (_KS2, np.uint32(_KS0 + np.uint32(5))),
)


def _threefry_bits(g):
  x1 = g + _KS1
  x0 = x1
  first = True
  for grp in range(5):
    for r in _ROTS[grp % 2]:
      if first:
        first = False
      else:
        x0 = x0 + x1
      x1 = ((x1 << np.uint32(r)) | (x1 >> np.uint32(32 - r))) ^ x0
    a, b = _INJECT[grp]
    x0 = x0 + a
    x1 = x1 + b
  return x0 ^ x1


def _uniform_from_g(g):
  bits = _threefry_bits(g)
  return lax.bitcast_convert_type(
      (bits >> np.uint32(9)) | np.uint32(0x3F800000), jnp.float32) - 1.0


def _sc_body(seq_hbm, out_rows, seq_v0, seq_v1, unk_v, row_ring, idx_ring,
             sem_seq, sem_sc):
  wid = lax.axis_index("s") * _NC + lax.axis_index("c")
  w_pos_base = wid * _ROWS_PER_W * _SEQ_LEN
  lanes = lax.iota(jnp.int32, 16)
  seq_bufs = (seq_v0, seq_v1)

  def seq_copy(c, b):
    return pltpu.make_async_copy(
        seq_hbm.at[pl.ds(w_pos_base + c * _CHUNK_POS, _CHUNK_POS)],
        seq_bufs[b], sem_seq.at[b])

  seq_copy(0, 0).start()
  seq_copy(1, 1).start()

  def process(c, b):
    seqc = seq_bufs[b]
    seq_copy(c, b).wait()

    def compact_body(gidx, cnt):
      s = seqc[pl.ds(gidx * 16, 16)]
      m = s == 24
      plsc.store_compressed(unk_v.at[pl.ds(cnt, 16)], gidx * 16 + lanes,
                            mask=m)
      return cnt + jnp.sum(jnp.where(m, jnp.int32(1), jnp.int32(0)))

    cnt = lax.fori_loop(0, _N_GRP, compact_body, jnp.int32(0))
    nh = (cnt + 15) >> 4
    chunk_pos_base = w_pos_base + c * _CHUNK_POS

    def do_group(h, r):
      sel = jnp.minimum(h * 16 + lanes, cnt - 1)
      posc = plsc.load_gather(unk_v, [sel])
      gpos = chunk_pos_base + posc
      gword = gpos * _NA
      gbase = gword.astype(jnp.uint32)
      rowbuf = row_ring.at[r]
      idxbuf = idx_ring.at[r]

      def k_body(j, ssum):
        k0 = j * 5
        for t in range(5):
          k = k0 + t
          woff = k * 16 + lanes
          u = _uniform_from_g(gbase + k.astype(jnp.uint32))
          plsc.store_scatter(rowbuf, [woff], u)
          plsc.store_scatter(idxbuf, [woff], gword + k)
          ssum = ssum + u
        return ssum

      ssum = lax.fori_loop(0, 5, k_body, jnp.zeros((16,), jnp.float32))
      inv = 1.0 / ssum

      def k_body2(j, _):
        k0 = j * 5
        for t in range(5):
          k = k0 + t
          woff = k * 16 + lanes
          u = plsc.load_gather(rowbuf, [woff])
          plsc.store_scatter(rowbuf, [woff], u * inv)
        return 0

      lax.fori_loop(0, 5, k_body2, jnp.int32(0))
      pltpu.make_async_copy(rowbuf, out_rows.at[idxbuf], sem_sc.at[r]).start()

    def wave_body(w, _):
      for r in range(_NSLOT):
        h = w * _NSLOT + r

        @pl.when(h < nh)
        def _():
          @pl.when(w >= 1)
          def _():
            pltpu.make_async_copy(
                row_ring.at[r], out_rows.at[idx_ring.at[r]],
                sem_sc.at[r]).wait()
          do_group(h, r)
      return 0

    lax.fori_loop(0, (nh + _NSLOT - 1) >> 2, wave_body, jnp.int32(0))

    for r in range(_NSLOT):
      @pl.when(r < nh)
      def _():
        pltpu.make_async_copy(
            row_ring.at[r], out_rows.at[idx_ring.at[r]], sem_sc.at[r]).wait()

    @pl.when(c + 2 < _N_CHUNKS)
    def _():
      seq_copy(c + 2, b).start()

  def outer(c2, carry):
    for b in range(2):
      process(c2 * 2 + b, b)
    return carry

  lax.fori_loop(0, _N_CHUNKS // 2, outer, jnp.int32(0))


@jax.jit
def kernel(seq):
  dense = pl.pallas_call(
      _tc_onehot_kernel,
      grid=(_N_ROWS // _TC_BLOCK_ROWS,),
      in_specs=[
          pl.BlockSpec((_TC_BLOCK_ROWS, _SEQ_LEN), lambda i: (i, 0)),
          pl.BlockSpec((_SEQ_LEN, _W), lambda i: (0, 0)),
      ],
      out_specs=pl.BlockSpec((_TC_BLOCK_ROWS, _W), lambda i: (i, 0)),
      out_shape=jax.ShapeDtypeStruct((_N_ROWS, _W), jnp.float32),
  )(seq, _REP_BF16)

  out_ref = jax.new_ref(dense.reshape(_N_POS * _NA))

  mesh = plsc.VectorSubcoreMesh(
      core_axis_name="c", subcore_axis_name="s", num_cores=_NC,
      num_subcores=_NS)
  sc_overwrite = pl.kernel(
      _sc_body,
      mesh=mesh,
      compiler_params=pltpu.CompilerParams(
          needs_layout_passes=False, use_tc_tiling_on_sc=False),
      scratch_types=[
          pltpu.VMEM((_CHUNK_POS,), jnp.int32),
          pltpu.VMEM((_CHUNK_POS,), jnp.int32),
          pltpu.VMEM((_CHUNK_POS,), jnp.int32),
          pltpu.VMEM((_NSLOT, _NA * 16), jnp.float32),
          pltpu.VMEM((_NSLOT, _NA * 16), jnp.int32),
          pltpu.SemaphoreType.DMA((2,)),
          pltpu.SemaphoreType.DMA((_NSLOT,)),
      ],
  )
  sc_overwrite(seq.reshape(_N_POS), out_ref)
  return out_ref[...].reshape(_N_ROWS, _SEQ_LEN, _NA)


# speedup vs baseline: 2.8531x; 2.8531x over previous
import functools

import numpy as np
import jax
import jax.numpy as jnp
from jax import lax
from jax.experimental import pallas as pl

_N_ROWS = 16384
_SEQ_LEN = 200
_NA = 25
_W = _SEQ_LEN * _NA

_KS0 = np.uint32(0)
_KS1 = np.uint32(42)
_KS2 = np.uint32(0x1BD11BDA ^ 42)
_ROTS = ((13, 15, 26, 6), (17, 29, 16, 24))
_INJECT = (
    (_KS1, np.uint32(_KS2 + np.uint32(1))),
    (_KS2, np.uint32(_KS0 + np.uint32(2))),
    (_KS0, np.uint32(_KS1 + np.uint32(3))),
    (_KS1, np.uint32(_KS2 + np.uint32(4))),
    (_KS2, np.uint32(_KS0 + np.uint32(5))),
)

_JDX = np.arange(_W) // _NA
_REP_NP = (_JDX[None, :] == np.arange(_SEQ_LEN)[:, None]).astype(np.float32)
_REP_BF16 = jnp.asarray(_REP_NP, dtype=jnp.bfloat16)
_SUM_BF16 = jnp.asarray(_REP_NP.T, dtype=jnp.bfloat16)


def _threefry_bits(g):
  x1 = g + _KS1
  x0 = x1
  first = True
  for grp in range(5):
    for r in _ROTS[grp % 2]:
      if first:
        first = False
      else:
        x0 = x0 + x1
      x1 = ((x1 << np.uint32(r)) | (x1 >> np.uint32(32 - r))) ^ x0
    a, b = _INJECT[grp]
    x0 = x0 + a
    x1 = x1 + b
  return x0 ^ x1


def _dense_kernel(seq_ref, rep_ref, sum_ref, out_ref, *, block_rows):
  pid = pl.program_id(0)
  qi = lax.broadcasted_iota(jnp.int32, (block_rows, _W), 1)
  ri = lax.broadcasted_iota(jnp.int32, (block_rows, _W), 0)

  base = (pid * block_rows * _W).astype(jnp.uint32)
  g = base + ri.astype(jnp.uint32) * np.uint32(_W) + qi.astype(jnp.uint32)
  bits = _threefry_bits(g)
  u = lax.bitcast_convert_type(
      (bits >> np.uint32(9)) | np.uint32(0x3F800000), jnp.float32) - 1.0

  seq_bf = seq_ref[...].astype(jnp.float32).astype(jnp.bfloat16)
  sval = jnp.dot(seq_bf, rep_ref[...], preferred_element_type=jnp.float32)

  rowsum = jnp.dot(u.astype(jnp.bfloat16), sum_ref[...],
                   preferred_element_type=jnp.float32)
  denom = jnp.dot(rowsum.astype(jnp.bfloat16), rep_ref[...],
                  preferred_element_type=jnp.float32)

  jdx = (qi * 10486) >> 18
  kdx = (qi - _NA * jdx).astype(jnp.float32)

  onehot = jnp.where(kdx == sval, 1.0, 0.0)
  out_ref[...] = jnp.where(sval == 24.0, u / denom, onehot)


@jax.jit
def kernel(seq):
  block_rows = 64
  out = pl.pallas_call(
      functools.partial(_dense_kernel, block_rows=block_rows),
      grid=(_N_ROWS // block_rows,),
      in_specs=[
          pl.BlockSpec((block_rows, _SEQ_LEN), lambda i: (i, 0)),
          pl.BlockSpec((_SEQ_LEN, _W), lambda i: (0, 0)),
          pl.BlockSpec((_W, _SEQ_LEN), lambda i: (0, 0)),
      ],
      out_specs=pl.BlockSpec((block_rows, _W), lambda i: (i, 0)),
      out_shape=jax.ShapeDtypeStruct((_N_ROWS, _W), jnp.float32),
  )(seq, _REP_BF16, _SUM_BF16)
  return out.reshape(_N_ROWS, _SEQ_LEN, _NA)


# speedup vs baseline: 3.0003x; 1.0516x over previous
import functools

import numpy as np
import jax
import jax.numpy as jnp
from jax import lax
from jax.experimental import pallas as pl

_N_ROWS = 16384
_SEQ_LEN = 200
_NA = 25
_W = _SEQ_LEN * _NA

_KS0 = np.uint32(0)
_KS1 = np.uint32(42)
_KS2 = np.uint32(0x1BD11BDA ^ 42)
_ROTS = ((13, 15, 26, 6), (17, 29, 16, 24))
_INJECT = (
    (_KS1, np.uint32(_KS2 + np.uint32(1))),
    (_KS2, np.uint32(_KS0 + np.uint32(2))),
    (_KS0, np.uint32(_KS1 + np.uint32(3))),
    (_KS1, np.uint32(_KS2 + np.uint32(4))),
    (_KS2, np.uint32(_KS0 + np.uint32(5))),
)

_JDX = np.arange(_W) // _NA
_REP_NP = (_JDX[None, :] == np.arange(_SEQ_LEN)[:, None]).astype(np.float32)
_REP_BF16 = jnp.asarray(_REP_NP, dtype=jnp.bfloat16)
_SUM_BF16 = jnp.asarray(_REP_NP.T, dtype=jnp.bfloat16)


def _threefry_bits(g):
  x1 = g + _KS1
  x0 = x1
  first = True
  for grp in range(5):
    for r in _ROTS[grp % 2]:
      if first:
        first = False
      else:
        x0 = x0 + x1
      x1 = ((x1 << np.uint32(r)) | (x1 >> np.uint32(32 - r))) ^ x0
    a, b = _INJECT[grp]
    x0 = x0 + a
    x1 = x1 + b
  return x0 ^ x1


def _dense_kernel(seq_ref, rep_ref, sum_ref, out_ref, *, block_rows):
  pid = pl.program_id(0)
  qi = lax.broadcasted_iota(jnp.int32, (block_rows, _W), 1)
  ri = lax.broadcasted_iota(jnp.int32, (block_rows, _W), 0)

  base = (pid * block_rows * _W).astype(jnp.uint32)
  g = base + ri.astype(jnp.uint32) * np.uint32(_W) + qi.astype(jnp.uint32)
  bits = _threefry_bits(g)
  u = lax.bitcast_convert_type(
      (bits >> np.uint32(9)) | np.uint32(0x3F800000), jnp.float32) - 1.0

  seq_bf = seq_ref[...].astype(jnp.float32).astype(jnp.bfloat16)
  sval = jnp.dot(seq_bf, rep_ref[...], preferred_element_type=jnp.float32)

  rowsum = jnp.dot(u.astype(jnp.bfloat16), sum_ref[...],
                   preferred_element_type=jnp.float32)
  denom = jnp.dot(rowsum.astype(jnp.bfloat16), rep_ref[...],
                  preferred_element_type=jnp.float32)

  jdx = (qi * 10486) >> 18
  kdx = (qi - _NA * jdx).astype(jnp.float32)

  onehot = jnp.where(kdx == sval, 1.0, 0.0)
  out_ref[...] = jnp.where(sval == 24.0, u / denom, onehot)


@jax.jit
def kernel(seq):
  block_rows = 128
  out = pl.pallas_call(
      functools.partial(_dense_kernel, block_rows=block_rows),
      grid=(_N_ROWS // block_rows,),
      in_specs=[
          pl.BlockSpec((block_rows, _SEQ_LEN), lambda i: (i, 0)),
          pl.BlockSpec((_SEQ_LEN, _W), lambda i: (0, 0)),
          pl.BlockSpec((_W, _SEQ_LEN), lambda i: (0, 0)),
      ],
      out_specs=pl.BlockSpec((block_rows, _W), lambda i: (i, 0)),
      out_shape=jax.ShapeDtypeStruct((_N_ROWS, _W), jnp.float32),
  )(seq, _REP_BF16, _SUM_BF16)
  return out.reshape(_N_ROWS, _SEQ_LEN, _NA)


# speedup vs baseline: 3.0164x; 1.0054x over previous
import functools

import numpy as np
import jax
import jax.numpy as jnp
from jax import lax
from jax.experimental import pallas as pl

_N_ROWS = 16384
_SEQ_LEN = 200
_NA = 25
_W = _SEQ_LEN * _NA

_KS0 = np.uint32(0)
_KS1 = np.uint32(42)
_KS2 = np.uint32(0x1BD11BDA ^ 42)
_ROTS = ((13, 15, 26, 6), (17, 29, 16, 24))
_INJECT = (
    (_KS1, np.uint32(_KS2 + np.uint32(1))),
    (_KS2, np.uint32(_KS0 + np.uint32(2))),
    (_KS0, np.uint32(_KS1 + np.uint32(3))),
    (_KS1, np.uint32(_KS2 + np.uint32(4))),
    (_KS2, np.uint32(_KS0 + np.uint32(5))),
)

_JDX = np.arange(_W) // _NA
_REP_NP = (_JDX[None, :] == np.arange(_SEQ_LEN)[:, None]).astype(np.float32)
_REP_BF16 = jnp.asarray(_REP_NP, dtype=jnp.bfloat16)
_SUM_BF16 = jnp.asarray(_REP_NP.T, dtype=jnp.bfloat16)


def _threefry_bits(g):
  x1 = g + _KS1
  x0 = x1
  first = True
  for grp in range(5):
    for r in _ROTS[grp % 2]:
      if first:
        first = False
      else:
        x0 = x0 + x1
      x1 = ((x1 << np.uint32(r)) | (x1 >> np.uint32(32 - r))) ^ x0
    a, b = _INJECT[grp]
    x0 = x0 + a
    x1 = x1 + b
  return x0 ^ x1


def _dense_kernel(seq_ref, rep_ref, sum_ref, out_ref, *, block_rows):
  pid = pl.program_id(0)
  qi = lax.broadcasted_iota(jnp.int32, (block_rows, _W), 1)
  ri = lax.broadcasted_iota(jnp.int32, (block_rows, _W), 0)

  base = (pid * block_rows * _W).astype(jnp.uint32)
  g = base + ri.astype(jnp.uint32) * np.uint32(_W) + qi.astype(jnp.uint32)
  bits = _threefry_bits(g)
  u = lax.bitcast_convert_type(
      (bits >> np.uint32(9)) | np.uint32(0x3F800000), jnp.float32) - 1.0

  seq_bf = seq_ref[...].astype(jnp.float32).astype(jnp.bfloat16)
  sval = jnp.dot(seq_bf, rep_ref[...], preferred_element_type=jnp.float32)

  rowsum = jnp.dot(u.astype(jnp.bfloat16), sum_ref[...],
                   preferred_element_type=jnp.float32)
  denom = jnp.dot(rowsum.astype(jnp.bfloat16), rep_ref[...],
                  preferred_element_type=jnp.float32)

  jdx = (qi * 10486) >> 18
  kdx = (qi - _NA * jdx).astype(jnp.float32)

  onehot = jnp.where(kdx == sval, 1.0, 0.0)
  out_ref[...] = jnp.where(sval == 24.0, u / denom, onehot)


@jax.jit
def kernel(seq):
  block_rows = 256
  out = pl.pallas_call(
      functools.partial(_dense_kernel, block_rows=block_rows),
      grid=(_N_ROWS // block_rows,),
      in_specs=[
          pl.BlockSpec((block_rows, _SEQ_LEN), lambda i: (i, 0)),
          pl.BlockSpec((_SEQ_LEN, _W), lambda i: (0, 0)),
          pl.BlockSpec((_W, _SEQ_LEN), lambda i: (0, 0)),
      ],
      out_specs=pl.BlockSpec((block_rows, _W), lambda i: (i, 0)),
      out_shape=jax.ShapeDtypeStruct((_N_ROWS, _W), jnp.float32),
  )(seq, _REP_BF16, _SUM_BF16)
  return out.reshape(_N_ROWS, _SEQ_LEN, _NA)
